# baseline (speedup 1.0000x reference)
import jax
import jax.numpy as jnp
from jax.experimental import pallas as pl

VINPUT = 1024
HIDDEN = 2048
TOPK = 2
NUM_EXPERTS = 8

_BM = 256


def _moe_dense_kernel(x_ref, gw_ref, w1_ref, b1_ref, w2_ref, b2_ref,
                      w3_ref, b3_ref, out_ref):
    e = pl.program_id(1)
    x = x_ref[...]
    logits = jnp.dot(x, gw_ref[...], preferred_element_type=jnp.float32)
    i1 = jnp.argmax(logits, axis=-1)
    iota = jax.lax.broadcasted_iota(jnp.int32, logits.shape, 1)
    mask1 = iota == i1[:, None]
    neg = jnp.float32(-jnp.inf)
    masked = jnp.where(mask1, neg, logits)
    i2 = jnp.argmax(masked, axis=-1)
    m1 = jnp.max(logits, axis=-1)
    m2 = jnp.max(masked, axis=-1)
    b = jnp.exp(m2 - m1)
    g1 = 1.0 / (1.0 + b)
    g2 = b / (1.0 + b)
    we = jnp.where(i1 == e, g1, 0.0) + jnp.where(i2 == e, g2, 0.0)

    h1 = jnp.dot(x, w1_ref[0], preferred_element_type=jnp.float32) + b1_ref[0]
    h2 = jnp.dot(x, w2_ref[0], preferred_element_type=jnp.float32) + b2_ref[0]
    h = h1 * (h2 * jax.nn.sigmoid(h2))
    y = jnp.dot(h, w3_ref[0], preferred_element_type=jnp.float32) + b3_ref[0]
    contrib = we[:, None] * y

    @pl.when(e == 0)
    def _init():
        out_ref[...] = contrib

    @pl.when(e != 0)
    def _acc():
        out_ref[...] += contrib


def kernel(x, gate_w, w1, b1, w2, b2, w3, b3):
    orig_shape = x.shape
    xf = x.reshape(-1, x.shape[-1])
    T = xf.shape[0]
    nb = T // _BM
    b1r = b1.reshape(NUM_EXPERTS, 1, HIDDEN)
    b2r = b2.reshape(NUM_EXPERTS, 1, HIDDEN)
    b3r = b3.reshape(NUM_EXPERTS, 1, VINPUT)
    out = pl.pallas_call(
        _moe_dense_kernel,
        grid=(nb, NUM_EXPERTS),
        in_specs=[
            pl.BlockSpec((_BM, VINPUT), lambda i, e: (i, 0)),
            pl.BlockSpec((VINPUT, NUM_EXPERTS), lambda i, e: (0, 0)),
            pl.BlockSpec((1, VINPUT, HIDDEN), lambda i, e: (e, 0, 0)),
            pl.BlockSpec((1, 1, HIDDEN), lambda i, e: (e, 0, 0)),
            pl.BlockSpec((1, VINPUT, HIDDEN), lambda i, e: (e, 0, 0)),
            pl.BlockSpec((1, 1, HIDDEN), lambda i, e: (e, 0, 0)),
            pl.BlockSpec((1, HIDDEN, VINPUT), lambda i, e: (e, 0, 0)),
            pl.BlockSpec((1, 1, VINPUT), lambda i, e: (e, 0, 0)),
        ],
        out_specs=pl.BlockSpec((_BM, VINPUT), lambda i, e: (i, 0)),
        out_shape=jax.ShapeDtypeStruct((T, VINPUT), jnp.float32),
    )(xf, gate_w, w1, b1r, w2, b2r, w3, b3r)
    return out.reshape(orig_shape)
